# baseline (speedup 1.0000x reference)
import functools

import jax
import jax.numpy as jnp
from jax import lax
from jax.experimental import pallas as pl
from jax.experimental.pallas import tpu as pltpu
from jax.experimental.pallas import tpu_sc as plsc

NC = 2
NS = 16
LANES = 16
EDGE_BATCH = 128


def _mm(x, w):
    n, kdim = x.shape
    dout = w.shape[0]
    br = 1000

    def body(x_ref, w_ref, o_ref):
        o_ref[...] = lax.dot_general(
            x_ref[...], w_ref[...], (((1,), (1,)), ((), ())),
            preferred_element_type=jnp.float32)

    return pl.pallas_call(
        body,
        grid=(n // br,),
        in_specs=[pl.BlockSpec((br, kdim), lambda i: (i, 0)),
                  pl.BlockSpec((dout, kdim), lambda i: (0, 0))],
        out_specs=pl.BlockSpec((br, dout), lambda i: (i, 0)),
        out_shape=jax.ShapeDtypeStruct((n, dout), jnp.float32),
    )(x, w)


def _fuse(part, w, gamma, beta, n):
    dk = part.shape[2]
    dout = w.shape[0]

    def body(p_ref, w_ref, g_ref, bt_ref, o_ref):
        s = p_ref[0] + p_ref[1]
        m = jnp.mean(s, axis=0, keepdims=True)
        c = s - m
        v = jnp.mean(c * c, axis=0, keepdims=True)
        h = g_ref[...] * c * lax.rsqrt(v + 1e-5) + bt_ref[...]
        h = jnp.maximum(h, 0.0)
        o_ref[...] = lax.dot_general(
            h, w_ref[...], (((1,), (1,)), ((), ())),
            preferred_element_type=jnp.float32)

    return pl.pallas_call(
        body,
        grid=(1,),
        in_specs=[pl.BlockSpec((2, n, dk), lambda i: (0, 0, 0)),
                  pl.BlockSpec((dout, dk), lambda i: (0, 0)),
                  pl.BlockSpec((1, dk), lambda i: (0, 0)),
                  pl.BlockSpec((1, dk), lambda i: (0, 0))],
        out_specs=pl.BlockSpec((n, dout), lambda i: (0, 0)),
        out_shape=jax.ShapeDtypeStruct((n, dout), jnp.float32),
    )(part, w, gamma.reshape(1, dk), beta.reshape(1, dk))


def _final_add(part, b3p, n, dout):
    dk = part.shape[2]

    def body(p_ref, b_ref, o_ref):
        o_ref[...] = (p_ref[0] + p_ref[1] + b_ref[...])[:, :dout]

    return pl.pallas_call(
        body,
        grid=(1,),
        in_specs=[pl.BlockSpec((2, n, dk), lambda i: (0, 0, 0)),
                  pl.BlockSpec((1, dk), lambda i: (0, 0))],
        out_specs=pl.BlockSpec((n, dout), lambda i: (0, 0)),
        out_shape=jax.ShapeDtypeStruct((n, dout), jnp.float32),
    )(part, b3p.reshape(1, dk))


@functools.lru_cache(maxsize=None)
def _make_spmm(n_pad, dk, nb):
    rpt = n_pad // NS
    nvec = dk // LANES
    ngrp = EDGE_BATCH // LANES
    NBUF = 4
    assert nb >= 2 * NBUF and nb % NBUF == 0
    mesh = plsc.VectorSubcoreMesh(core_axis_name="c", subcore_axis_name="s")

    @functools.partial(
        pl.kernel,
        out_type=jax.ShapeDtypeStruct((NC, n_pad, dk), jnp.float32),
        mesh=mesh,
        compiler_params=pltpu.CompilerParams(
            use_tc_tiling_on_sc=False, needs_layout_passes=False),
        scratch_types=[
            pltpu.VMEM_SHARED((n_pad, dk), jnp.float32),
            pltpu.VMEM_SHARED((n_pad, dk), jnp.float32),
            pltpu.VMEM((NBUF, 4, EDGE_BATCH), jnp.int32),
            pltpu.VMEM((NBUF, EDGE_BATCH, dk), jnp.float32),
            pltpu.VMEM((NBUF, EDGE_BATCH), jnp.int32),
        ] + [pltpu.SemaphoreType.DMA] * (3 * NBUF),
    )
    def spmm(h_hbm, edata_hbm, zero_hbm, out_hbm,
             acc, hs, e_v, rows, dcp, *sems):
        cid = lax.axis_index("c")
        sid = lax.axis_index("s")
        wid = cid * NS + sid
        se = sems[:NBUF]
        sg = sems[NBUF:2 * NBUF]
        ss = sems[2 * NBUF:]
        n_rows = h_hbm.shape[0]
        last_h = n_rows - (NS - 1) * rpt

        pltpu.sync_copy(zero_hbm.at[pl.ds(sid * rpt, rpt)],
                        acc.at[pl.ds(sid * rpt, rpt)])

        @pl.when(sid < NS - 1)
        def _():
            pltpu.sync_copy(h_hbm.at[pl.ds(sid * rpt, rpt)],
                            hs.at[pl.ds(sid * rpt, rpt)])

        @pl.when(sid == NS - 1)
        def _():
            pltpu.sync_copy(h_hbm.at[pl.ds((NS - 1) * rpt, last_h)],
                            hs.at[pl.ds((NS - 1) * rpt, last_h)])

        plsc.subcore_barrier()

        def gather_start(m):
            pltpu.async_copy(hs.at[e_v.at[m].at[0]], rows.at[m], sg[m])

        def gather_wait(m):
            pltpu.make_async_copy(
                hs.at[e_v.at[m].at[0]], rows.at[m], sg[m]).wait()

        def edata_start(m, bb):
            pltpu.async_copy(edata_hbm.at[wid, bb], e_v.at[m], se[m])

        def edata_wait(m):
            pltpu.make_async_copy(
                edata_hbm.at[wid, 0], e_v.at[m], se[m]).wait()

        def scat_wait(m):
            pltpu.make_async_copy(rows.at[m], acc.at[dcp.at[m]],
                                  ss[m]).wait()

        def step(bb, m, do_scat_wait=True):
            m1 = (m + 1) % NBUF
            edata_wait(m1)
            if do_scat_wait:
                scat_wait(m1)
            gather_start(m1)
            gather_wait(m)
            em = e_v.at[m]
            rm = rows.at[m]
            dm = dcp.at[m]
            for t in range(ngrp):
                sl = pl.ds(t * LANES, LANES)
                dm[sl] = em[1, sl]

            @pl.loop(0, ngrp)
            def _grp(g):
                vvv = plsc.bitcast(em[2, pl.ds(g * LANES, LANES)],
                                   jnp.float32)
                for i in range(0, LANES, 2):
                    j0 = g * LANES + i
                    j1 = g * LANES + i + 1
                    v0 = vvv[i]
                    v1 = vvv[i + 1]
                    for k in range(nvec):
                        sl = pl.ds(k * LANES, LANES)
                        a = rm[j0, sl] * v0
                        b = rm[j1, sl] * v1
                        rm[j0, sl] = a
                        rm[j1, sl] = b

            pltpu.async_copy(rm, acc.at[dm], ss[m], add=True)
            edata_start(m, bb + NBUF)

        pltpu.async_copy(edata_hbm.at[wid, 0], e_v.at[0], se[0]).wait()
        gather_start(0)
        for m in range(1, NBUF):
            edata_start(m, m)
        for bb in range(NBUF - 1):
            step(bb, bb, do_scat_wait=False)

        @pl.loop(NBUF - 1, nb - 1, step=NBUF)
        def _quad(b):
            for ph in range(NBUF):
                step(b + ph, (NBUF - 1 + ph) % NBUF)

        step(nb - 1, (nb - 1) % NBUF)

        for m in range(1, NBUF):
            edata_wait(m)
        gather_wait(0)
        for m in range(1, NBUF):
            scat_wait(m)

        plsc.subcore_barrier()
        pltpu.sync_copy(acc.at[pl.ds(sid * rpt, rpt)],
                        out_hbm.at[cid, pl.ds(sid * rpt, rpt)])

    return spmm


def kernel(x, adj_indices, adj_values, W1, b1, gamma1, beta1,
           W2, b2, gamma2, beta2, W3, b3):
    n = x.shape[0]
    hid = W1.shape[0]
    dlast = 16
    e = adj_values.shape[0]
    group = NC * NS * EDGE_BATCH
    e_pad = ((e + group - 1) // group) * group
    pad = e_pad - e
    n_pad = ((n + NS * 8 - 1) // (NS * 8)) * (NS * 8)

    dst = jnp.concatenate([adj_indices[0], jnp.zeros((pad,), jnp.int32)])
    src = jnp.concatenate([adj_indices[1], jnp.zeros((pad,), jnp.int32)])
    val = jnp.concatenate([adj_values, jnp.zeros((pad,), jnp.float32)])
    nw = NC * NS
    epw = e_pad // nw
    nb = epw // EDGE_BATCH

    def _tile(a):
        a = a.reshape(nw, epw)
        a = jnp.concatenate(
            [a, jnp.zeros((nw, 4 * EDGE_BATCH), jnp.int32)], axis=1)
        return a.reshape(nw, nb + 4, EDGE_BATCH)

    edata = jnp.stack(
        [_tile(src), _tile(dst), _tile(lax.bitcast_convert_type(val, jnp.int32)),
         _tile(jnp.zeros((e_pad,), jnp.int32))], axis=2)

    zhid = jnp.zeros((n_pad, hid), jnp.float32)
    zlast = jnp.zeros((n_pad, dlast), jnp.float32)
    w3p = jnp.zeros((dlast, hid), jnp.float32).at[:W3.shape[0], :].set(W3)
    b3p = jnp.zeros((dlast,), jnp.float32).at[:W3.shape[0]].set(b3)

    spmm_h = _make_spmm(n_pad, hid, nb)
    spmm_l = _make_spmm(n_pad, dlast, nb)

    y1 = _mm(x, W1)
    p1 = spmm_h(y1, edata, zhid)
    y2 = _fuse(p1, W2, gamma1, beta1, n)
    p2 = spmm_h(y2, edata, zhid)
    y3 = _fuse(p2, w3p, gamma2, beta2, n)
    p3 = spmm_l(y3, edata, zlast)
    return _final_add(p3, b3p, n, W3.shape[0])


# speedup vs baseline: 1.0017x; 1.0017x over previous
import functools

import jax
import jax.numpy as jnp
from jax import lax
from jax.experimental import pallas as pl
from jax.experimental.pallas import tpu as pltpu
from jax.experimental.pallas import tpu_sc as plsc

NC = 2
NS = 16
LANES = 16
EDGE_BATCH = 128


def _mm(x, w):
    n, kdim = x.shape
    dout = w.shape[0]
    br = 1000

    def body(x_ref, w_ref, o_ref):
        o_ref[...] = lax.dot_general(
            x_ref[...], w_ref[...], (((1,), (1,)), ((), ())),
            preferred_element_type=jnp.float32)

    return pl.pallas_call(
        body,
        grid=(n // br,),
        in_specs=[pl.BlockSpec((br, kdim), lambda i: (i, 0)),
                  pl.BlockSpec((dout, kdim), lambda i: (0, 0))],
        out_specs=pl.BlockSpec((br, dout), lambda i: (i, 0)),
        out_shape=jax.ShapeDtypeStruct((n, dout), jnp.float32),
    )(x, w)


def _fuse(part, w, gamma, beta, n):
    dk = part.shape[2]
    dout = w.shape[0]

    def body(p_ref, w_ref, g_ref, bt_ref, o_ref):
        s = p_ref[0] + p_ref[1]
        m = jnp.mean(s, axis=0, keepdims=True)
        c = s - m
        v = jnp.mean(c * c, axis=0, keepdims=True)
        h = g_ref[...] * c * lax.rsqrt(v + 1e-5) + bt_ref[...]
        h = jnp.maximum(h, 0.0)
        o_ref[...] = lax.dot_general(
            h, w_ref[...], (((1,), (1,)), ((), ())),
            preferred_element_type=jnp.float32)

    return pl.pallas_call(
        body,
        grid=(1,),
        in_specs=[pl.BlockSpec((2, n, dk), lambda i: (0, 0, 0)),
                  pl.BlockSpec((dout, dk), lambda i: (0, 0)),
                  pl.BlockSpec((1, dk), lambda i: (0, 0)),
                  pl.BlockSpec((1, dk), lambda i: (0, 0))],
        out_specs=pl.BlockSpec((n, dout), lambda i: (0, 0)),
        out_shape=jax.ShapeDtypeStruct((n, dout), jnp.float32),
    )(part, w, gamma.reshape(1, dk), beta.reshape(1, dk))


def _final_add(part, b3p, n, dout):
    dk = part.shape[2]

    def body(p_ref, b_ref, o_ref):
        o_ref[...] = (p_ref[0] + p_ref[1] + b_ref[...])[:, :dout]

    return pl.pallas_call(
        body,
        grid=(1,),
        in_specs=[pl.BlockSpec((2, n, dk), lambda i: (0, 0, 0)),
                  pl.BlockSpec((1, dk), lambda i: (0, 0))],
        out_specs=pl.BlockSpec((n, dout), lambda i: (0, 0)),
        out_shape=jax.ShapeDtypeStruct((n, dout), jnp.float32),
    )(part, b3p.reshape(1, dk))


@functools.lru_cache(maxsize=None)
def _make_spmm(n_pad, dk, nb):
    rpt = n_pad // NS
    nvec = dk // LANES
    ngrp = EDGE_BATCH // LANES
    NBUF = 4
    assert nb >= 2 * NBUF and nb % NBUF == 0
    mesh = plsc.VectorSubcoreMesh(core_axis_name="c", subcore_axis_name="s")

    @functools.partial(
        pl.kernel,
        out_type=jax.ShapeDtypeStruct((NC, n_pad, dk), jnp.float32),
        mesh=mesh,
        compiler_params=pltpu.CompilerParams(
            use_tc_tiling_on_sc=False, needs_layout_passes=False),
        scratch_types=[
            pltpu.VMEM_SHARED((n_pad, dk), jnp.float32),
            pltpu.VMEM_SHARED((n_pad, dk), jnp.float32),
            pltpu.VMEM((NBUF, 3, EDGE_BATCH), jnp.int32),
            pltpu.VMEM((NBUF, EDGE_BATCH, dk), jnp.float32),
            pltpu.VMEM((NBUF, EDGE_BATCH), jnp.int32),
        ] + [pltpu.SemaphoreType.DMA] * (3 * NBUF),
    )
    def spmm(h_hbm, edata_hbm, zero_hbm, out_hbm,
             acc, hs, e_v, rows, dcp, *sems):
        cid = lax.axis_index("c")
        sid = lax.axis_index("s")
        wid = cid * NS + sid
        se = sems[:NBUF]
        sg = sems[NBUF:2 * NBUF]
        ss = sems[2 * NBUF:]
        n_rows = h_hbm.shape[0]
        last_h = n_rows - (NS - 1) * rpt

        pltpu.sync_copy(zero_hbm.at[pl.ds(sid * rpt, rpt)],
                        acc.at[pl.ds(sid * rpt, rpt)])

        @pl.when(sid < NS - 1)
        def _():
            pltpu.sync_copy(h_hbm.at[pl.ds(sid * rpt, rpt)],
                            hs.at[pl.ds(sid * rpt, rpt)])

        @pl.when(sid == NS - 1)
        def _():
            pltpu.sync_copy(h_hbm.at[pl.ds((NS - 1) * rpt, last_h)],
                            hs.at[pl.ds((NS - 1) * rpt, last_h)])

        plsc.subcore_barrier()

        def gather_start(m):
            pltpu.async_copy(hs.at[e_v.at[m].at[0]], rows.at[m], sg[m])

        def gather_wait(m):
            pltpu.make_async_copy(
                hs.at[e_v.at[m].at[0]], rows.at[m], sg[m]).wait()

        def edata_start(m, bb):
            pltpu.async_copy(edata_hbm.at[wid, bb], e_v.at[m], se[m])

        def edata_wait(m):
            pltpu.make_async_copy(
                edata_hbm.at[wid, 0], e_v.at[m], se[m]).wait()

        def scat_wait(m):
            pltpu.make_async_copy(rows.at[m], acc.at[dcp.at[m]],
                                  ss[m]).wait()

        def step(bb, m, do_scat_wait=True):
            m1 = (m + 1) % NBUF
            edata_wait(m1)
            if do_scat_wait:
                scat_wait(m1)
            gather_start(m1)
            gather_wait(m)
            em = e_v.at[m]
            rm = rows.at[m]
            dm = dcp.at[m]
            for t in range(ngrp):
                sl = pl.ds(t * LANES, LANES)
                dm[sl] = em[1, sl]

            @pl.loop(0, ngrp)
            def _grp(g):
                vvv = plsc.bitcast(em[2, pl.ds(g * LANES, LANES)],
                                   jnp.float32)
                for i in range(0, LANES, 2):
                    j0 = g * LANES + i
                    j1 = g * LANES + i + 1
                    v0 = vvv[i]
                    v1 = vvv[i + 1]
                    for k in range(nvec):
                        sl = pl.ds(k * LANES, LANES)
                        a = rm[j0, sl] * v0
                        b = rm[j1, sl] * v1
                        rm[j0, sl] = a
                        rm[j1, sl] = b

            pltpu.async_copy(rm, acc.at[dm], ss[m], add=True)
            edata_start(m, bb + NBUF)

        pltpu.async_copy(edata_hbm.at[wid, 0], e_v.at[0], se[0]).wait()
        gather_start(0)
        for m in range(1, NBUF):
            edata_start(m, m)
        for bb in range(NBUF - 1):
            step(bb, bb, do_scat_wait=False)

        @pl.loop(NBUF - 1, nb - 1, step=NBUF)
        def _quad(b):
            for ph in range(NBUF):
                step(b + ph, (NBUF - 1 + ph) % NBUF)

        step(nb - 1, (nb - 1) % NBUF)

        for m in range(1, NBUF):
            edata_wait(m)
        gather_wait(0)
        for m in range(1, NBUF):
            scat_wait(m)

        plsc.subcore_barrier()
        pltpu.sync_copy(acc.at[pl.ds(sid * rpt, rpt)],
                        out_hbm.at[cid, pl.ds(sid * rpt, rpt)])

    return spmm


def kernel(x, adj_indices, adj_values, W1, b1, gamma1, beta1,
           W2, b2, gamma2, beta2, W3, b3):
    n = x.shape[0]
    hid = W1.shape[0]
    dlast = 16
    e = adj_values.shape[0]
    group = NC * NS * EDGE_BATCH
    e_pad = ((e + group - 1) // group) * group
    pad = e_pad - e
    n_pad = ((n + NS * 8 - 1) // (NS * 8)) * (NS * 8)

    dst = jnp.concatenate([adj_indices[0], jnp.zeros((pad,), jnp.int32)])
    src = jnp.concatenate([adj_indices[1], jnp.zeros((pad,), jnp.int32)])
    val = jnp.concatenate([adj_values, jnp.zeros((pad,), jnp.float32)])
    nw = NC * NS
    epw = e_pad // nw
    nb = epw // EDGE_BATCH

    def _tile(a):
        a = a.reshape(nw, epw)
        a = jnp.concatenate(
            [a, jnp.zeros((nw, 4 * EDGE_BATCH), jnp.int32)], axis=1)
        return a.reshape(nw, nb + 4, EDGE_BATCH)

    edata = jnp.stack(
        [_tile(src), _tile(dst),
         _tile(lax.bitcast_convert_type(val, jnp.int32))], axis=2)

    zhid = jnp.zeros((n_pad, hid), jnp.float32)
    zlast = jnp.zeros((n_pad, dlast), jnp.float32)
    w3p = jnp.zeros((dlast, hid), jnp.float32).at[:W3.shape[0], :].set(W3)
    b3p = jnp.zeros((dlast,), jnp.float32).at[:W3.shape[0]].set(b3)

    spmm_h = _make_spmm(n_pad, hid, nb)
    spmm_l = _make_spmm(n_pad, dlast, nb)

    y1 = _mm(x, W1)
    p1 = spmm_h(y1, edata, zhid)
    y2 = _fuse(p1, W2, gamma1, beta1, n)
    p2 = spmm_h(y2, edata, zhid)
    y3 = _fuse(p2, w3p, gamma2, beta2, n)
    p3 = spmm_l(y3, edata, zlast)
    return _final_add(p3, b3p, n, W3.shape[0])


# speedup vs baseline: 1.1461x; 1.1441x over previous
import functools

import jax
import jax.numpy as jnp
from jax import lax
from jax.experimental import pallas as pl
from jax.experimental.pallas import tpu as pltpu
from jax.experimental.pallas import tpu_sc as plsc

NC = 2
NS = 16
LANES = 16
EDGE_BATCH = 128


def _mm(x, w):
    n, kdim = x.shape
    dout = w.shape[0]
    br = 1000

    def body(x_ref, w_ref, o_ref):
        o_ref[...] = lax.dot_general(
            x_ref[...], w_ref[...], (((1,), (1,)), ((), ())),
            preferred_element_type=jnp.float32)

    return pl.pallas_call(
        body,
        grid=(n // br,),
        in_specs=[pl.BlockSpec((br, kdim), lambda i: (i, 0)),
                  pl.BlockSpec((dout, kdim), lambda i: (0, 0))],
        out_specs=pl.BlockSpec((br, dout), lambda i: (i, 0)),
        out_shape=jax.ShapeDtypeStruct((n, dout), jnp.float32),
    )(x, w)


def _fuse(part, wb, gamma, beta, n, dk):
    dout2 = wb.shape[1]
    nh = n // 2

    def body(p_ref, w_ref, g_ref, bt_ref, o_ref):
        s = p_ref[0] + p_ref[1]
        cs = jnp.sum(s, axis=0, keepdims=True)
        sq = jnp.sum(s * s, axis=0, keepdims=True)
        m = (cs[:, :dk] + cs[:, dk:]) * (1.0 / n)
        ex2 = (sq[:, :dk] + sq[:, dk:]) * (1.0 / n)
        v = ex2 - m * m
        gs = g_ref[...] * lax.rsqrt(v + 1e-5)
        off = bt_ref[...] - gs * m
        gs2 = jnp.concatenate([gs, gs], axis=1)
        off2 = jnp.concatenate([off, off], axis=1)
        h = jnp.maximum(gs2 * s + off2, 0.0)
        o_ref[...] = lax.dot_general(
            h, w_ref[...], (((1,), (0,)), ((), ())),
            preferred_element_type=jnp.float32)

    return pl.pallas_call(
        body,
        grid=(1,),
        in_specs=[pl.BlockSpec((2, nh, 128), lambda i: (0, 0, 0)),
                  pl.BlockSpec((128, dout2), lambda i: (0, 0)),
                  pl.BlockSpec((1, dk), lambda i: (0, 0)),
                  pl.BlockSpec((1, dk), lambda i: (0, 0))],
        out_specs=pl.BlockSpec((nh, dout2), lambda i: (0, 0)),
        out_shape=jax.ShapeDtypeStruct((nh, dout2), jnp.float32),
    )(part, wb, gamma.reshape(1, dk), beta.reshape(1, dk))


def _final_add(part, b3p, n, dout):
    dk = part.shape[2]

    def body(p_ref, b_ref, o_ref):
        o_ref[...] = (p_ref[0] + p_ref[1] + b_ref[...])[:, :dout]

    return pl.pallas_call(
        body,
        grid=(1,),
        in_specs=[pl.BlockSpec((2, n, dk), lambda i: (0, 0, 0)),
                  pl.BlockSpec((1, dk), lambda i: (0, 0))],
        out_specs=pl.BlockSpec((n, dout), lambda i: (0, 0)),
        out_shape=jax.ShapeDtypeStruct((n, dout), jnp.float32),
    )(part, b3p.reshape(1, dk))


@functools.lru_cache(maxsize=None)
def _make_spmm(n_pad, dk, nb):
    rpt = n_pad // NS
    nvec = dk // LANES
    ngrp = EDGE_BATCH // LANES
    NBUF = 4
    assert nb >= 2 * NBUF and nb % NBUF == 0
    mesh = plsc.VectorSubcoreMesh(core_axis_name="c", subcore_axis_name="s")

    @functools.partial(
        pl.kernel,
        out_type=jax.ShapeDtypeStruct((NC, n_pad, dk), jnp.float32),
        mesh=mesh,
        compiler_params=pltpu.CompilerParams(
            use_tc_tiling_on_sc=False, needs_layout_passes=False),
        scratch_types=[
            pltpu.VMEM_SHARED((n_pad, dk), jnp.float32),
            pltpu.VMEM_SHARED((n_pad, dk), jnp.float32),
            pltpu.VMEM((NBUF, 3, EDGE_BATCH), jnp.int32),
            pltpu.VMEM((NBUF, EDGE_BATCH, dk), jnp.float32),
            pltpu.VMEM((NBUF, EDGE_BATCH), jnp.int32),
        ] + [pltpu.SemaphoreType.DMA] * (3 * NBUF),
    )
    def spmm(h_hbm, edata_hbm, zero_hbm, out_hbm,
             acc, hs, e_v, rows, dcp, *sems):
        cid = lax.axis_index("c")
        sid = lax.axis_index("s")
        wid = cid * NS + sid
        se = sems[:NBUF]
        sg = sems[NBUF:2 * NBUF]
        ss = sems[2 * NBUF:]
        n_rows = h_hbm.shape[0]
        last_h = n_rows - (NS - 1) * rpt

        pltpu.sync_copy(zero_hbm.at[pl.ds(sid * rpt, rpt)],
                        acc.at[pl.ds(sid * rpt, rpt)])

        @pl.when(sid < NS - 1)
        def _():
            pltpu.sync_copy(h_hbm.at[pl.ds(sid * rpt, rpt)],
                            hs.at[pl.ds(sid * rpt, rpt)])

        @pl.when(sid == NS - 1)
        def _():
            pltpu.sync_copy(h_hbm.at[pl.ds((NS - 1) * rpt, last_h)],
                            hs.at[pl.ds((NS - 1) * rpt, last_h)])

        plsc.subcore_barrier()

        def gather_start(m):
            pltpu.async_copy(hs.at[e_v.at[m].at[0]], rows.at[m], sg[m])

        def gather_wait(m):
            pltpu.make_async_copy(
                hs.at[e_v.at[m].at[0]], rows.at[m], sg[m]).wait()

        def edata_start(m, bb):
            pltpu.async_copy(edata_hbm.at[wid, bb], e_v.at[m], se[m])

        def edata_wait(m):
            pltpu.make_async_copy(
                edata_hbm.at[wid, 0], e_v.at[m], se[m]).wait()

        def scat_wait(m):
            pltpu.make_async_copy(rows.at[m], acc.at[dcp.at[m]],
                                  ss[m]).wait()

        def step(bb, m, do_scat_wait=True):
            m1 = (m + 1) % NBUF
            edata_wait(m1)
            if do_scat_wait:
                scat_wait(m1)
            gather_start(m1)
            gather_wait(m)
            em = e_v.at[m]
            rm = rows.at[m]
            dm = dcp.at[m]
            for t in range(ngrp):
                sl = pl.ds(t * LANES, LANES)
                dm[sl] = em[1, sl]

            @pl.loop(0, ngrp)
            def _grp(g):
                vvv = plsc.bitcast(em[2, pl.ds(g * LANES, LANES)],
                                   jnp.float32)
                for i in range(0, LANES, 2):
                    j0 = g * LANES + i
                    j1 = g * LANES + i + 1
                    v0 = vvv[i]
                    v1 = vvv[i + 1]
                    for k in range(nvec):
                        sl = pl.ds(k * LANES, LANES)
                        a = rm[j0, sl] * v0
                        b = rm[j1, sl] * v1
                        rm[j0, sl] = a
                        rm[j1, sl] = b

            pltpu.async_copy(rm, acc.at[dm], ss[m], add=True)
            edata_start(m, bb + NBUF)

        pltpu.async_copy(edata_hbm.at[wid, 0], e_v.at[0], se[0]).wait()
        gather_start(0)
        for m in range(1, NBUF):
            edata_start(m, m)
        for bb in range(NBUF - 1):
            step(bb, bb, do_scat_wait=False)

        @pl.loop(NBUF - 1, nb - 1, step=NBUF)
        def _quad(b):
            for ph in range(NBUF):
                step(b + ph, (NBUF - 1 + ph) % NBUF)

        step(nb - 1, (nb - 1) % NBUF)

        for m in range(1, NBUF):
            edata_wait(m)
        gather_wait(0)
        for m in range(1, NBUF):
            scat_wait(m)

        plsc.subcore_barrier()
        pltpu.sync_copy(acc.at[pl.ds(sid * rpt, rpt)],
                        out_hbm.at[cid, pl.ds(sid * rpt, rpt)])

    return spmm


def kernel(x, adj_indices, adj_values, W1, b1, gamma1, beta1,
           W2, b2, gamma2, beta2, W3, b3):
    n = x.shape[0]
    hid = W1.shape[0]
    dlast = 16
    e = adj_values.shape[0]
    group = NC * NS * EDGE_BATCH
    e_pad = ((e + group - 1) // group) * group
    pad = e_pad - e
    n_pad = ((n + NS * 8 - 1) // (NS * 8)) * (NS * 8)

    dst = jnp.concatenate([adj_indices[0], jnp.zeros((pad,), jnp.int32)])
    src = jnp.concatenate([adj_indices[1], jnp.zeros((pad,), jnp.int32)])
    val = jnp.concatenate([adj_values, jnp.zeros((pad,), jnp.float32)])
    nw = NC * NS
    epw = e_pad // nw
    nb = epw // EDGE_BATCH

    def _tile(a):
        a = a.reshape(nw, epw)
        a = jnp.concatenate(
            [a, jnp.zeros((nw, 4 * EDGE_BATCH), jnp.int32)], axis=1)
        return a.reshape(nw, nb + 4, EDGE_BATCH)

    edata = jnp.stack(
        [_tile(src), _tile(dst),
         _tile(lax.bitcast_convert_type(val, jnp.int32))], axis=2)

    zhid = jnp.zeros((n_pad, hid), jnp.float32)
    zlast = jnp.zeros((n_pad, dlast), jnp.float32)
    w3p = jnp.zeros((dlast, hid), jnp.float32).at[:W3.shape[0], :].set(W3)
    b3p = jnp.zeros((dlast,), jnp.float32).at[:W3.shape[0]].set(b3)
    w2b = (jnp.zeros((2 * hid, 2 * hid), jnp.float32)
           .at[:hid, :hid].set(W2.T).at[hid:, hid:].set(W2.T))
    w3b = (jnp.zeros((2 * hid, 2 * dlast), jnp.float32)
           .at[:hid, :dlast].set(w3p.T).at[hid:, dlast:].set(w3p.T))

    spmm_h = _make_spmm(n_pad, hid, nb)
    spmm_l = _make_spmm(n_pad, dlast, nb)

    y1 = _mm(x, W1)
    p1 = spmm_h(y1, edata, zhid)
    y2 = _fuse(p1.reshape(2, n_pad // 2, 128),
               w2b, gamma1, beta1, n, hid)
    p2 = spmm_h(y2.reshape(n, hid), edata, zhid)
    y3 = _fuse(p2.reshape(2, n_pad // 2, 128),
               w3b, gamma2, beta2, n, hid)
    p3 = spmm_l(y3.reshape(n, dlast), edata, zlast)
    return _final_add(p3, b3p, n, W3.shape[0])


# speedup vs baseline: 1.1730x; 1.0235x over previous
import functools

import jax
import jax.numpy as jnp
from jax import lax
from jax.experimental import pallas as pl
from jax.experimental.pallas import tpu as pltpu
from jax.experimental.pallas import tpu_sc as plsc

NC = 2
NS = 16
LANES = 16
EDGE_BATCH = 128


def _mm(x, w):
    n, kdim = x.shape
    dout = w.shape[0]
    br = 1000

    def body(x_ref, w_ref, o_ref):
        o_ref[...] = lax.dot_general(
            x_ref[...], w_ref[...], (((1,), (1,)), ((), ())),
            preferred_element_type=jnp.float32)

    return pl.pallas_call(
        body,
        grid=(n // br,),
        in_specs=[pl.BlockSpec((br, kdim), lambda i: (i, 0)),
                  pl.BlockSpec((dout, kdim), lambda i: (0, 0))],
        out_specs=pl.BlockSpec((br, dout), lambda i: (i, 0)),
        out_shape=jax.ShapeDtypeStruct((n, dout), jnp.float32),
    )(x, w)


def _fuse(part, wb, gamma, beta, n, dk):
    dout2 = wb.shape[1]
    nh = n // 2

    def body(p_ref, w_ref, g_ref, bt_ref, o_ref):
        s = p_ref[0] + p_ref[1]
        cs = jnp.sum(s, axis=0, keepdims=True)
        sq = jnp.sum(s * s, axis=0, keepdims=True)
        m = (cs[:, :dk] + cs[:, dk:]) * (1.0 / n)
        ex2 = (sq[:, :dk] + sq[:, dk:]) * (1.0 / n)
        v = ex2 - m * m
        gs = g_ref[...] * lax.rsqrt(v + 1e-5)
        off = bt_ref[...] - gs * m
        gs2 = jnp.concatenate([gs, gs], axis=1)
        off2 = jnp.concatenate([off, off], axis=1)
        h = jnp.maximum(gs2 * s + off2, 0.0)
        o_ref[...] = lax.dot_general(
            h, w_ref[...], (((1,), (0,)), ((), ())),
            preferred_element_type=jnp.float32)

    return pl.pallas_call(
        body,
        grid=(1,),
        in_specs=[pl.BlockSpec((2, nh, 128), lambda i: (0, 0, 0)),
                  pl.BlockSpec((128, dout2), lambda i: (0, 0)),
                  pl.BlockSpec((1, dk), lambda i: (0, 0)),
                  pl.BlockSpec((1, dk), lambda i: (0, 0))],
        out_specs=pl.BlockSpec((nh, dout2), lambda i: (0, 0)),
        out_shape=jax.ShapeDtypeStruct((nh, dout2), jnp.float32),
    )(part, wb, gamma.reshape(1, dk), beta.reshape(1, dk))


def _final_add(part, b3t):
    np8 = part.shape[1]

    def body(p_ref, b_ref, o_ref):
        o_ref[...] = p_ref[0] + p_ref[1] + b_ref[...]

    return pl.pallas_call(
        body,
        grid=(1,),
        in_specs=[pl.BlockSpec((2, np8, 128), lambda i: (0, 0, 0)),
                  pl.BlockSpec((1, 128), lambda i: (0, 0))],
        out_specs=pl.BlockSpec((np8, 128), lambda i: (0, 0)),
        out_shape=jax.ShapeDtypeStruct((np8, 128), jnp.float32),
    )(part, b3t.reshape(1, 128))


@functools.lru_cache(maxsize=None)
def _make_spmm(n_pad, dk, nb):
    rpt = n_pad // NS
    nvec = dk // LANES
    ngrp = EDGE_BATCH // LANES
    NBUF = 4
    assert nb >= 2 * NBUF and nb % NBUF == 0
    mesh = plsc.VectorSubcoreMesh(core_axis_name="c", subcore_axis_name="s")

    @functools.partial(
        pl.kernel,
        out_type=jax.ShapeDtypeStruct((NC, n_pad, dk), jnp.float32),
        mesh=mesh,
        compiler_params=pltpu.CompilerParams(
            use_tc_tiling_on_sc=False, needs_layout_passes=False),
        scratch_types=[
            pltpu.VMEM_SHARED((n_pad, dk), jnp.float32),
            pltpu.VMEM_SHARED((n_pad, dk), jnp.float32),
            pltpu.VMEM((NBUF, 3, EDGE_BATCH), jnp.int32),
            pltpu.VMEM((NBUF, EDGE_BATCH, dk), jnp.float32),
            pltpu.VMEM((NBUF, EDGE_BATCH), jnp.int32),
        ] + [pltpu.SemaphoreType.DMA] * (3 * NBUF),
    )
    def spmm(h_hbm, edata_hbm, zero_hbm, out_hbm,
             acc, hs, e_v, rows, dcp, *sems):
        cid = lax.axis_index("c")
        sid = lax.axis_index("s")
        wid = cid * NS + sid
        se = sems[:NBUF]
        sg = sems[NBUF:2 * NBUF]
        ss = sems[2 * NBUF:]
        n_rows = h_hbm.shape[0]
        last_h = n_rows - (NS - 1) * rpt

        pltpu.sync_copy(zero_hbm.at[pl.ds(sid * rpt, rpt)],
                        acc.at[pl.ds(sid * rpt, rpt)])

        @pl.when(sid < NS - 1)
        def _():
            pltpu.sync_copy(h_hbm.at[pl.ds(sid * rpt, rpt)],
                            hs.at[pl.ds(sid * rpt, rpt)])

        @pl.when(sid == NS - 1)
        def _():
            pltpu.sync_copy(h_hbm.at[pl.ds((NS - 1) * rpt, last_h)],
                            hs.at[pl.ds((NS - 1) * rpt, last_h)])

        plsc.subcore_barrier()

        def gather_start(m):
            pltpu.async_copy(hs.at[e_v.at[m].at[0]], rows.at[m], sg[m])

        def gather_wait(m):
            pltpu.make_async_copy(
                hs.at[e_v.at[m].at[0]], rows.at[m], sg[m]).wait()

        def edata_start(m, bb):
            pltpu.async_copy(edata_hbm.at[wid, bb], e_v.at[m], se[m])

        def edata_wait(m):
            pltpu.make_async_copy(
                edata_hbm.at[wid, 0], e_v.at[m], se[m]).wait()

        def scat_wait(m):
            pltpu.make_async_copy(rows.at[m], acc.at[dcp.at[m]],
                                  ss[m]).wait()

        def step(bb, m, do_scat_wait=True):
            m1 = (m + 1) % NBUF
            edata_wait(m1)
            if do_scat_wait:
                scat_wait(m1)
            gather_start(m1)
            gather_wait(m)
            em = e_v.at[m]
            rm = rows.at[m]
            dm = dcp.at[m]
            for t in range(ngrp):
                sl = pl.ds(t * LANES, LANES)
                dm[sl] = em[1, sl]

            @pl.loop(0, ngrp)
            def _grp(g):
                vvv = plsc.bitcast(em[2, pl.ds(g * LANES, LANES)],
                                   jnp.float32)
                for i in range(0, LANES, 2):
                    j0 = g * LANES + i
                    j1 = g * LANES + i + 1
                    v0 = vvv[i]
                    v1 = vvv[i + 1]
                    for k in range(nvec):
                        sl = pl.ds(k * LANES, LANES)
                        a = rm[j0, sl] * v0
                        b = rm[j1, sl] * v1
                        rm[j0, sl] = a
                        rm[j1, sl] = b

            pltpu.async_copy(rm, acc.at[dm], ss[m], add=True)
            edata_start(m, bb + NBUF)

        pltpu.async_copy(edata_hbm.at[wid, 0], e_v.at[0], se[0]).wait()
        gather_start(0)
        for m in range(1, NBUF):
            edata_start(m, m)
        for bb in range(NBUF - 1):
            step(bb, bb, do_scat_wait=False)

        @pl.loop(NBUF - 1, nb - 1, step=NBUF)
        def _quad(b):
            for ph in range(NBUF):
                step(b + ph, (NBUF - 1 + ph) % NBUF)

        step(nb - 1, (nb - 1) % NBUF)

        for m in range(1, NBUF):
            edata_wait(m)
        gather_wait(0)
        for m in range(1, NBUF):
            scat_wait(m)

        plsc.subcore_barrier()
        pltpu.sync_copy(acc.at[pl.ds(sid * rpt, rpt)],
                        out_hbm.at[cid, pl.ds(sid * rpt, rpt)])

    return spmm


def kernel(x, adj_indices, adj_values, W1, b1, gamma1, beta1,
           W2, b2, gamma2, beta2, W3, b3):
    n = x.shape[0]
    hid = W1.shape[0]
    dlast = 16
    e = adj_values.shape[0]
    group = NC * NS * EDGE_BATCH
    e_pad = ((e + group - 1) // group) * group
    pad = e_pad - e
    n_pad = ((n + NS * 8 - 1) // (NS * 8)) * (NS * 8)

    dst = jnp.concatenate([adj_indices[0], jnp.zeros((pad,), jnp.int32)])
    src = jnp.concatenate([adj_indices[1], jnp.zeros((pad,), jnp.int32)])
    val = jnp.concatenate([adj_values, jnp.zeros((pad,), jnp.float32)])
    nw = NC * NS
    epw = e_pad // nw
    nb = epw // EDGE_BATCH

    def _tile(a):
        a = a.reshape(nw, epw)
        a = jnp.concatenate(
            [a, jnp.zeros((nw, 4 * EDGE_BATCH), jnp.int32)], axis=1)
        return a.reshape(nw, nb + 4, EDGE_BATCH)

    edata = jnp.stack(
        [_tile(src), _tile(dst),
         _tile(lax.bitcast_convert_type(val, jnp.int32))], axis=2)

    zhid = jnp.zeros((n_pad, hid), jnp.float32)
    zlast = jnp.zeros((n_pad, dlast), jnp.float32)
    w3p = jnp.zeros((dlast, hid), jnp.float32).at[:W3.shape[0], :].set(W3)
    b3p = jnp.zeros((dlast,), jnp.float32).at[:W3.shape[0]].set(b3)
    w2b = (jnp.zeros((2 * hid, 2 * hid), jnp.float32)
           .at[:hid, :hid].set(W2.T).at[hid:, hid:].set(W2.T))
    w3b = (jnp.zeros((2 * hid, 2 * dlast), jnp.float32)
           .at[:hid, :dlast].set(w3p.T).at[hid:, dlast:].set(w3p.T))

    spmm_h = _make_spmm(n_pad, hid, nb)
    spmm_l = _make_spmm(n_pad, dlast, nb)

    y1 = _mm(x, W1)
    p1 = spmm_h(y1, edata, zhid)
    y2 = _fuse(p1.reshape(2, n_pad // 2, 128),
               w2b, gamma1, beta1, n, hid)
    p2 = spmm_h(y2.reshape(n, hid), edata, zhid)
    y3 = _fuse(p2.reshape(2, n_pad // 2, 128),
               w3b, gamma2, beta2, n, hid)
    p3 = spmm_l(y3.reshape(n, dlast), edata, zlast)
    b3t = jnp.tile(b3p, 128 // dlast)
    out = _final_add(p3.reshape(2, n_pad * dlast // 128, 128), b3t)
    return out.reshape(n_pad, dlast)[:n, :W3.shape[0]]
